# baseline (speedup 1.0000x reference)
import jax
import jax.numpy as jnp
from jax.experimental import pallas as pl
from jax.experimental.pallas import tpu as pltpu

NN_ = 7
SX_ = 27
SX0_ = 23
SH_ = 1024
SZ_ = 256


def _body(x_ref, mf_ref, mb_ref, dg_ref,
          wcih, wlih, wrih,
          wchh, wlhh, wrhh,
          g1, g2, m1, m2,
          wmu, wstd,
          bcih, bchh, blih, blhh, brih, brhh,
          bg, bmu, bstd,
          out_ref, c10_a, c01_a, d_a, c10_b, c01_b, d_b):
    f32 = jnp.float32
    H = SH_

    def mmb(a, w_ref):
        return jnp.dot(a.astype(jnp.bfloat16), w_ref[...],
                       preferred_element_type=f32)

    def gru_combine(gi, gh, h_prev):
        r = jax.nn.sigmoid(gi[:, :H] + gh[:, :H])
        z = jax.nn.sigmoid(gi[:, H:2 * H] + gh[:, H:2 * H])
        n = jnp.tanh(gi[:, 2 * H:] + r * gh[:, 2 * H:])
        return n + z * (h_prev - n)

    BB = out_ref.shape[0]
    HB = BB // 2
    chains = ((slice(0, HB), (c10_a, c01_a, d_a)),
              (slice(HB, BB), (c10_b, c01_b, d_b)))

    def emit(v, scr, h_u):
        c10_s, c01_s, d_s = scr
        a1 = mmb(h_u, g1)
        a2 = mmb(h_u, g2)
        p1 = mmb(h_u, m1)
        p2 = mmb(h_u, m2)
        bgv = bg[...]
        c10 = jax.nn.sigmoid(a1 + bgv) * p1
        c01 = jax.nn.sigmoid(a2 + bgv) * p2
        c11 = jax.nn.sigmoid(a1 + a2 + bgv) * (p1 + p2)
        c10_s[v] = c10
        c01_s[v] = c01.astype(jnp.bfloat16)
        d_s[v] = (c11 - c10 - c01).astype(jnp.bfloat16)

    def gather(v, sl, scr):
        c10_s, c01_s, d_s = scr
        mf_slab = mf_ref[v, sl]
        mb_slab = mb_ref[v, sl]
        h_in = jnp.zeros((HB, H), f32)
        for u in range(1, NN_):
            mf = mf_slab[:, u:u + 1] > 0.0
            mb = mb_slab[:, u:u + 1] > 0.0
            h_in = (h_in
                    + jnp.where(mf, c10_s[u], 0.0)
                    + jnp.where(mb, c01_s[u], jnp.bfloat16(0.0))
                    + jnp.where(jnp.logical_and(mf, mb), d_s[u],
                                jnp.bfloat16(0.0)))
        return h_in

    def node6(sl, scr):
        x6 = x_ref[NN_ - 1, sl]
        gi = mmb(x6, wcih) + bcih[...]
        gh = jnp.broadcast_to(bchh[...], gi.shape)
        h = gru_combine(gi, gh, 0.0)
        ml = dg_ref[NN_ - 1, sl]
        gi2 = ml * mmb(x6, wlih) + blih[...]
        gh2 = mmb(h, wlhh) + blhh[...]
        h = gru_combine(gi2, gh2, h)
        emit(NN_ - 1, scr, h)

    for sl, scr in chains:
        node6(sl, scr)

    def node_mid(v, sl, scr):
        xv = x_ref[v, sl]
        h_in = gather(v, sl, scr)
        gi = mmb(xv, wcih) + bcih[...]
        gh = mmb(h_in, wchh) + bchh[...]
        hv = gru_combine(gi, gh, h_in)
        ml = dg_ref[v, sl]
        gi2 = ml * mmb(xv, wlih) + blih[...]
        gh2 = mmb(hv, wlhh) + blhh[...]
        hv = gru_combine(gi2, gh2, hv)
        emit(v, scr, hv)

    def node_step(t, carry):
        v = NN_ - 2 - t
        for sl, scr in chains:
            node_mid(v, sl, scr)
        return carry

    jax.lax.fori_loop(0, NN_ - 2, node_step, 0)

    def root(sl, scr):
        x0 = x_ref[0, sl][:, :SX0_]
        h_in = gather(0, sl, scr)
        gi = mmb(x0, wrih) + brih[...]
        gh = mmb(h_in, wrhh) + brhh[...]
        h0 = gru_combine(gi, gh, h_in)
        mu = mmb(h0, wmu) + bmu[...]
        sd = mmb(h0, wstd) + bstd[...]
        out_ref[sl] = jnp.concatenate([mu, jax.nn.softplus(sd)], axis=1)

    for sl, scr in chains:
        root(sl, scr)


def kernel(X, adj, params):
    f32, bf16 = jnp.float32, jnp.bfloat16
    B = X.shape[0]
    BB = 256 if B % 256 == 0 else B
    p = params
    adjf = adj.astype(f32)

    xt = jnp.transpose(X, (1, 0, 2)).astype(bf16)
    tri = (jnp.arange(NN_)[None, None, :]
           > jnp.arange(NN_)[:, None, None]).astype(f32)
    mfg = (jnp.transpose(adjf, (2, 0, 1)) * tri).astype(bf16)
    mbg = (jnp.transpose(adjf, (1, 0, 2)) * tri).astype(bf16)
    diag = jnp.transpose(
        adjf[:, jnp.arange(NN_), jnp.arange(NN_)])[..., None].astype(bf16)

    gate_w = p["gate_W"]
    map_w = p["map_W"]
    ins = (
        xt, mfg, mbg, diag,
        p["combin_Wih"].T.astype(bf16), p["loop_Wih"].T.astype(bf16),
        p["root_Wih"].T.astype(bf16),
        p["combin_Whh"].T.astype(bf16), p["loop_Whh"].T.astype(bf16),
        p["root_Whh"].T.astype(bf16),
        gate_w[:, :SH_].T.astype(bf16), gate_w[:, SH_:].T.astype(bf16),
        map_w[:, :SH_].T.astype(bf16), map_w[:, SH_:].T.astype(bf16),
        p["mu_W"].T.astype(bf16), p["std_W"].T.astype(bf16),
        p["combin_bih"].reshape(1, -1), p["combin_bhh"].reshape(1, -1),
        p["loop_bih"].reshape(1, -1), p["loop_bhh"].reshape(1, -1),
        p["root_bih"].reshape(1, -1), p["root_bhh"].reshape(1, -1),
        p["gate_b"].reshape(1, -1),
        p["mu_b"].reshape(1, -1), p["std_b"].reshape(1, -1),
    )

    def full_spec(a):
        nd = a.ndim
        return pl.BlockSpec(a.shape, lambda i, _nd=nd: (0,) * _nd)

    in_specs = [
        pl.BlockSpec((NN_, BB, SX_), lambda i: (0, i, 0)),
        pl.BlockSpec((NN_, BB, NN_), lambda i: (0, i, 0)),
        pl.BlockSpec((NN_, BB, NN_), lambda i: (0, i, 0)),
        pl.BlockSpec((NN_, BB, 1), lambda i: (0, i, 0)),
    ] + [full_spec(a) for a in ins[4:]]

    return pl.pallas_call(
        _body,
        grid=(B // BB,),
        in_specs=in_specs,
        out_specs=pl.BlockSpec((BB, 2 * SZ_), lambda i: (i, 0)),
        out_shape=jax.ShapeDtypeStruct((B, 2 * SZ_), f32),
        scratch_shapes=[
            pltpu.VMEM((NN_, BB // 2, SH_), f32),
            pltpu.VMEM((NN_, BB // 2, SH_), bf16),
            pltpu.VMEM((NN_, BB // 2, SH_), bf16),
            pltpu.VMEM((NN_, BB // 2, SH_), f32),
            pltpu.VMEM((NN_, BB // 2, SH_), bf16),
            pltpu.VMEM((NN_, BB // 2, SH_), bf16),
        ],
        compiler_params=pltpu.CompilerParams(
            dimension_semantics=("arbitrary",),
        ),
    )(*ins)


# speedup vs baseline: 1.0508x; 1.0508x over previous
import jax
import jax.numpy as jnp
from jax.experimental import pallas as pl
from jax.experimental.pallas import tpu as pltpu

NN_ = 7
SX_ = 27
SX0_ = 23
SH_ = 1024
SZ_ = 256


def _body(x_ref, mf_ref, mb_ref, dg_ref,
          wx, wrih,
          wchh, wlhh, wrhh,
          wem,
          wmu, wstd,
          bcih, bchh, blih, blhh, brih, brhh,
          bg, bmu, bstd,
          out_ref, c10_s, c01_s, d_s):
    f32 = jnp.float32
    H = SH_
    BB = out_ref.shape[0]

    def mmb(a, w_ref):
        return jnp.dot(a.astype(jnp.bfloat16), w_ref[...],
                       preferred_element_type=f32)

    def gru_combine(gi, gh, h_prev):
        r = jax.nn.sigmoid(gi[:, :H] + gh[:, :H])
        z = jax.nn.sigmoid(gi[:, H:2 * H] + gh[:, H:2 * H])
        n = jnp.tanh(gi[:, 2 * H:] + r * gh[:, 2 * H:])
        return n + z * (h_prev - n)

    def emit(v, h_u):
        e = mmb(h_u, wem)
        a1 = e[:, :H]
        a2 = e[:, H:2 * H]
        p1 = e[:, 2 * H:3 * H]
        p2 = e[:, 3 * H:]
        bgv = bg[...]
        c10 = jax.nn.sigmoid(a1 + bgv) * p1
        c01 = jax.nn.sigmoid(a2 + bgv) * p2
        c11 = jax.nn.sigmoid(a1 + a2 + bgv) * (p1 + p2)
        c10_s[v] = c10
        c01_s[v] = c01.astype(jnp.bfloat16)
        d_s[v] = (c11 - c10 - c01).astype(jnp.bfloat16)

    def gather(v):
        mf_slab = mf_ref[v]
        mb_slab = mb_ref[v]
        h_in = jnp.zeros((BB, H), f32)
        for u in range(1, NN_):
            mf = mf_slab[:, u:u + 1] > 0.0
            mb = mb_slab[:, u:u + 1] > 0.0
            h_in = (h_in
                    + jnp.where(mf, c10_s[u], 0.0)
                    + jnp.where(mb, c01_s[u], jnp.bfloat16(0.0))
                    + jnp.where(jnp.logical_and(mf, mb), d_s[u],
                                jnp.bfloat16(0.0)))
        return h_in

    x6 = x_ref[NN_ - 1]
    gx = mmb(x6, wx)
    gi = gx[:, :3 * H] + bcih[...]
    gh = jnp.broadcast_to(bchh[...], gi.shape)
    h = gru_combine(gi, gh, 0.0)
    ml = dg_ref[NN_ - 1]
    gi2 = ml * gx[:, 3 * H:] + blih[...]
    gh2 = mmb(h, wlhh) + blhh[...]
    h = gru_combine(gi2, gh2, h)
    emit(NN_ - 1, h)

    def node_step(t, carry):
        v = NN_ - 2 - t
        xv = x_ref[v]
        gx = mmb(xv, wx)
        h_in = gather(v)
        gi = gx[:, :3 * H] + bcih[...]
        gh = mmb(h_in, wchh) + bchh[...]
        hv = gru_combine(gi, gh, h_in)
        ml = dg_ref[v]
        gi2 = ml * gx[:, 3 * H:] + blih[...]
        gh2 = mmb(hv, wlhh) + blhh[...]
        hv = gru_combine(gi2, gh2, hv)
        emit(v, hv)
        return carry

    jax.lax.fori_loop(0, NN_ - 2, node_step, 0)

    x0 = x_ref[0][:, :SX0_]
    h_in = gather(0)
    gi = mmb(x0, wrih) + brih[...]
    gh = mmb(h_in, wrhh) + brhh[...]
    h0 = gru_combine(gi, gh, h_in)

    mu = mmb(h0, wmu) + bmu[...]
    sd = mmb(h0, wstd) + bstd[...]
    out_ref[...] = jnp.concatenate([mu, jax.nn.softplus(sd)], axis=1)


def kernel(X, adj, params):
    f32, bf16 = jnp.float32, jnp.bfloat16
    B = X.shape[0]
    BB = 256 if B % 256 == 0 else B
    p = params
    adjf = adj.astype(f32)

    xt = jnp.transpose(X, (1, 0, 2)).astype(bf16)
    tri = (jnp.arange(NN_)[None, None, :]
           > jnp.arange(NN_)[:, None, None]).astype(f32)
    mfg = (jnp.transpose(adjf, (2, 0, 1)) * tri).astype(bf16)
    mbg = (jnp.transpose(adjf, (1, 0, 2)) * tri).astype(bf16)
    diag = jnp.transpose(
        adjf[:, jnp.arange(NN_), jnp.arange(NN_)])[..., None].astype(bf16)

    gate_w = p["gate_W"]
    map_w = p["map_W"]
    wx = jnp.concatenate([p["combin_Wih"].T, p["loop_Wih"].T],
                         axis=1).astype(bf16)
    wem = jnp.concatenate(
        [gate_w[:, :SH_].T, gate_w[:, SH_:].T,
         map_w[:, :SH_].T, map_w[:, SH_:].T], axis=1).astype(bf16)
    ins = (
        xt, mfg, mbg, diag,
        wx, p["root_Wih"].T.astype(bf16),
        p["combin_Whh"].T.astype(bf16), p["loop_Whh"].T.astype(bf16),
        p["root_Whh"].T.astype(bf16),
        wem,
        p["mu_W"].T.astype(bf16), p["std_W"].T.astype(bf16),
        p["combin_bih"].reshape(1, -1), p["combin_bhh"].reshape(1, -1),
        p["loop_bih"].reshape(1, -1), p["loop_bhh"].reshape(1, -1),
        p["root_bih"].reshape(1, -1), p["root_bhh"].reshape(1, -1),
        p["gate_b"].reshape(1, -1),
        p["mu_b"].reshape(1, -1), p["std_b"].reshape(1, -1),
    )

    def full_spec(a):
        nd = a.ndim
        return pl.BlockSpec(a.shape, lambda i, _nd=nd: (0,) * _nd)

    in_specs = [
        pl.BlockSpec((NN_, BB, SX_), lambda i: (0, i, 0)),
        pl.BlockSpec((NN_, BB, NN_), lambda i: (0, i, 0)),
        pl.BlockSpec((NN_, BB, NN_), lambda i: (0, i, 0)),
        pl.BlockSpec((NN_, BB, 1), lambda i: (0, i, 0)),
    ] + [full_spec(a) for a in ins[4:]]

    return pl.pallas_call(
        _body,
        grid=(B // BB,),
        in_specs=in_specs,
        out_specs=pl.BlockSpec((BB, 2 * SZ_), lambda i: (i, 0)),
        out_shape=jax.ShapeDtypeStruct((B, 2 * SZ_), f32),
        scratch_shapes=[
            pltpu.VMEM((NN_, BB, SH_), f32),
            pltpu.VMEM((NN_, BB, SH_), bf16),
            pltpu.VMEM((NN_, BB, SH_), bf16),
        ],
        compiler_params=pltpu.CompilerParams(
            dimension_semantics=("arbitrary",),
        ),
    )(*ins)
